# Initial kernel scaffold; baseline (speedup 1.0000x reference)
import jax
import jax.numpy as jnp
from jax.experimental import pallas as pl


def kernel(x, rbf, sbf, edge_idx_kj, edge_idx_ji, W_rbf1, W_rbf2, W_sbf1, W_sbf2, W_kj, b_kj, W_ji, b_ji, W_down, W_up, W_bs1a, b_bs1a, W_bs1b, b_bs1b, W_bs2, b_bs2, W_as1a, b_as1a, W_as1b, b_as1b, W_as2a, b_as2a, W_as2b, b_as2b):
    raise NotImplementedError("write your pallas kernel here")


# speedup vs baseline: 1.9511x; 1.9511x over previous
import jax
import jax.numpy as jnp
import numpy as np
from jax import lax
from jax.experimental import pallas as pl
from jax.experimental.pallas import tpu as pltpu
from jax.experimental.pallas import tpu_sc as plsc

NE = 160000
NT = 800000
D = 128
DD = 64

NC = 2
NS = 16
NW = NC * NS

CH = 1280
JB = 10
NCHUNK = NT // CH

W_STARTS = (0, 53504, 107008)
W_SIZES = (53504, 53504, 52992)
DUMP_BASE = 53504
ACC_ROWS = 53760
ZSTRIPE = ACC_ROWS // NS

BF = jnp.bfloat16
F32 = jnp.float32
I32 = jnp.int32


def _swish(v):
    return v * jax.nn.sigmoid(v)


def _bf(v):
    return v.astype(BF)


def _pack_pairs(vals_bf):
    bits = lax.bitcast_convert_type(vals_bf, jnp.uint16).astype(I32)
    packed = []
    for hh in range(2):
        lo = bits[:, 32 * hh:32 * hh + 16]
        hi = bits[:, 32 * hh + 16:32 * hh + 32]
        packed.append(lo | (hi << 16))
    return jnp.stack(packed, axis=0)


def _head_body(x_ref, rbf_ref, wkj_ref, bkj_ref, w1_ref, w2_ref, wd_ref,
               out_ref):
    xb = x_ref[...]
    t = _swish(jnp.dot(_bf(xb), _bf(wkj_ref[...]),
                       preferred_element_type=F32) + bkj_ref[...])
    r1 = jnp.dot(rbf_ref[...], w1_ref[...], preferred_element_type=F32)
    re = jnp.dot(r1, w2_ref[...], preferred_element_type=F32)
    xkj = t * re
    xd = _swish(jnp.dot(_bf(xkj), _bf(wd_ref[...]),
                        preferred_element_type=F32))
    out_ref[...] = _pack_pairs(xd.astype(BF))


def _tc_head(x, rbf, w_kj, b_kj, w_rbf1, w_rbf2, w_down):
    bn = 2000
    grid = NE // bn
    nr = rbf.shape[1]
    be = w_rbf1.shape[1]
    return pl.pallas_call(
        _head_body,
        grid=(grid,),
        in_specs=[
            pl.BlockSpec((bn, D), lambda i: (i, 0)),
            pl.BlockSpec((bn, nr), lambda i: (i, 0)),
            pl.BlockSpec((D, D), lambda i: (0, 0)),
            pl.BlockSpec((1, D), lambda i: (0, 0)),
            pl.BlockSpec((nr, be), lambda i: (0, 0)),
            pl.BlockSpec((be, D), lambda i: (0, 0)),
            pl.BlockSpec((D, DD), lambda i: (0, 0)),
        ],
        out_specs=pl.BlockSpec((2, bn, 16), lambda i: (0, i, 0)),
        out_shape=jax.ShapeDtypeStruct((2, NE, 16), I32),
    )(x, rbf, w_kj, b_kj.reshape(1, D), w_rbf1, w_rbf2, w_down)


def _sbf_body(sbf_ref, w1_ref, w2_ref, out_ref):
    s1 = jnp.dot(sbf_ref[...], w1_ref[...], preferred_element_type=F32)
    se = jnp.dot(s1, w2_ref[...], preferred_element_type=F32)
    out_ref[...] = _pack_pairs(se.astype(BF))


def _tc_sbf(sbf, w_sbf1, w_sbf2):
    bt = 8000
    grid = NT // bt
    k = sbf.shape[1]
    be = w_sbf1.shape[1]
    return pl.pallas_call(
        _sbf_body,
        grid=(grid,),
        in_specs=[
            pl.BlockSpec((bt, k), lambda i: (i, 0)),
            pl.BlockSpec((k, be), lambda i: (0, 0)),
            pl.BlockSpec((be, DD), lambda i: (0, 0)),
        ],
        out_specs=pl.BlockSpec((2, bt, 16), lambda i: (0, i, 0)),
        out_shape=jax.ShapeDtypeStruct((2, NT, 16), I32),
    )(sbf, w_sbf1, w_sbf2)


_HIMASK = -65536


def _sc_body(kj_hbm, ji_hbm, xd_hbm, sbfe_hbm, out_hbm,
             kjv, jiv, didxv, rows_i, sbfev_i, prod, zbuf, acc,
             sem_g, sem_sc):
    c = lax.axis_index("c")
    s = lax.axis_index("s")
    wid = s * NC + c

    zeros16 = jnp.zeros((16,), F32)
    for r in range(zbuf.shape[0]):
        zbuf[r, pl.ds(0, 16)] = zeros16
        zbuf[r, pl.ds(16, 16)] = zeros16

    n_i = (NCHUNK - 1 - wid) // NW + 1
    zrows = zbuf.shape[0]

    for w in range(3):
        for h in range(2):
            w0 = W_STARTS[w]
            w1 = w0 + W_SIZES[w]

            zb = s * ZSTRIPE
            for t in range(ZSTRIPE // zrows):
                pltpu.sync_copy(zbuf, acc.at[pl.ds(zb + t * zrows, zrows)])
            plsc.subcore_barrier()

            def chunk_body(i, carry):
                cid = wid + NW * i
                pltpu.sync_copy(kj_hbm.at[cid], kjv)
                pltpu.sync_copy(ji_hbm.at[cid], jiv)

                def fire(j):
                    b = j % 2
                    g = pltpu.async_copy(
                        xd_hbm.at[h].at[kjv.at[j]], rows_i.at[b], sem_g)
                    sb = pltpu.async_copy(
                        sbfe_hbm.at[h, cid, pl.ds(j * 128, 128)],
                        sbfev_i.at[b], sem_g)
                    return (g, sb)

                gd = {0: fire(0)}
                for j in range(JB):
                    for l in range(8):
                        sl = (j, pl.ds(l * 16, 16))
                        ji = jiv[sl]
                        inw = (ji >= w0) & (ji < w1)
                        dump = DUMP_BASE + (ji & 63)
                        didxv[sl] = jnp.where(inw, ji - w0, dump)

                sd = {}
                for j in range(JB):
                    b = j % 2
                    if j + 1 < JB:
                        gd[j + 1] = fire(j + 1)
                    for d in gd.pop(j):
                        d.wait()
                    if j - 2 in sd:
                        sd.pop(j - 2).wait()

                    def mul_body(r4, carry2):
                        base = pl.multiple_of(r4 * 4, 4)
                        for rr in range(4):
                            r = base + rr
                            vx = rows_i[b, r, pl.ds(0, 16)]
                            vs = sbfev_i[b, r, pl.ds(0, 16)]
                            xlo = lax.bitcast_convert_type(vx << 16, F32)
                            xhi = lax.bitcast_convert_type(vx & _HIMASK, F32)
                            slo = lax.bitcast_convert_type(vs << 16, F32)
                            shi = lax.bitcast_convert_type(vs & _HIMASK, F32)
                            prod[b, r, pl.ds(0, 16)] = xlo * slo
                            prod[b, r, pl.ds(16, 16)] = xhi * shi
                        return carry2
                    lax.fori_loop(0, 128 // 4, mul_body, 0)

                    sd[j] = pltpu.async_copy(
                        prod.at[b], acc.at[didxv.at[j]], sem_sc, add=True)
                for d in sd.values():
                    d.wait()
                return carry

            lax.fori_loop(0, n_i, chunk_body, 0)
            plsc.subcore_barrier()

            fs = W_SIZES[w] // NS
            fb = s * fs
            pltpu.sync_copy(
                acc.at[pl.ds(fb, fs)],
                out_hbm.at[c, h, pl.ds(w0 + fb, fs)])
            plsc.subcore_barrier()


def _sc_scatter(kj3, ji3, xd2, sbfe3i):
    mesh = plsc.VectorSubcoreMesh(core_axis_name="c", subcore_axis_name="s")
    f = pl.kernel(
        _sc_body,
        out_type=jax.ShapeDtypeStruct((NC, 2, NE, 32), F32),
        mesh=mesh,
        compiler_params=pltpu.CompilerParams(use_tc_tiling_on_sc=False),
        scratch_types=[
            pltpu.VMEM((JB, 128), I32),
            pltpu.VMEM((JB, 128), I32),
            pltpu.VMEM((JB, 128), I32),
            pltpu.VMEM((2, 128, 16), I32),
            pltpu.VMEM((2, 128, 16), I32),
            pltpu.VMEM((2, 128, 32), F32),
            pltpu.VMEM((80, 32), F32),
            pltpu.VMEM_SHARED((ACC_ROWS, 32), F32),
            pltpu.SemaphoreType.DMA,
            pltpu.SemaphoreType.DMA,
        ],
    )
    return f(kj3, ji3, xd2, sbfe3i)


def _tail_body(x_ref, p_ref, wup_ref, wji_ref, bji_ref,
               wbs1a_ref, bbs1a_ref, wbs1b_ref, bbs1b_ref,
               wbs2_ref, bbs2_ref,
               was1a_ref, bas1a_ref, was1b_ref, bas1b_ref,
               was2a_ref, bas2a_ref, was2b_ref, bas2b_ref,
               out_ref):
    xb = x_ref[...]
    agg = jnp.concatenate(
        [p_ref[0, 0] + p_ref[1, 0], p_ref[0, 1] + p_ref[1, 1]], axis=-1)
    xkj = _swish(jnp.dot(_bf(agg), _bf(wup_ref[...]),
                         preferred_element_type=F32))
    xji = _swish(jnp.dot(_bf(xb), _bf(wji_ref[...]),
                         preferred_element_type=F32) + bji_ref[...])

    def res(hh, wa_ref, ba_ref, wb_ref, bb_ref):
        t1 = _swish(jnp.dot(_bf(hh), _bf(wa_ref[...]),
                            preferred_element_type=F32) + ba_ref[...])
        t2 = _swish(jnp.dot(_bf(t1), _bf(wb_ref[...]),
                            preferred_element_type=F32) + bb_ref[...])
        return hh + t2

    h = res(xji + xkj, wbs1a_ref, bbs1a_ref, wbs1b_ref, bbs1b_ref)
    h = _swish(jnp.dot(_bf(h), _bf(wbs2_ref[...]),
                       preferred_element_type=F32) + bbs2_ref[...]) + xb
    h = res(h, was1a_ref, bas1a_ref, was1b_ref, bas1b_ref)
    h = res(h, was2a_ref, bas2a_ref, was2b_ref, bas2b_ref)
    out_ref[...] = h


def _tc_tail(x, parts, w_up, w_ji, b_ji, weights):
    bn = 2000
    grid = NE // bn
    wspec = pl.BlockSpec((D, D), lambda i: (0, 0))
    bspec = pl.BlockSpec((1, D), lambda i: (0, 0))
    wb = []
    specs = []
    for (wmat, bvec) in weights:
        wb.append(wmat)
        wb.append(bvec.reshape(1, D))
        specs.append(wspec)
        specs.append(bspec)
    return pl.pallas_call(
        _tail_body,
        grid=(grid,),
        in_specs=[
            pl.BlockSpec((bn, D), lambda i: (i, 0)),
            pl.BlockSpec((NC, 2, bn, 32), lambda i: (0, 0, i, 0)),
            pl.BlockSpec((DD, D), lambda i: (0, 0)),
            wspec,
            bspec,
        ] + specs,
        out_specs=pl.BlockSpec((bn, D), lambda i: (i, 0)),
        out_shape=jax.ShapeDtypeStruct((NE, D), F32),
    )(x, parts, w_up, w_ji, b_ji.reshape(1, D), *wb)


def kernel(x, rbf, sbf, edge_idx_kj, edge_idx_ji, W_rbf1, W_rbf2, W_sbf1,
           W_sbf2, W_kj, b_kj, W_ji, b_ji, W_down, W_up, W_bs1a, b_bs1a,
           W_bs1b, b_bs1b, W_bs2, b_bs2, W_as1a, b_as1a, W_as1b, b_as1b,
           W_as2a, b_as2a, W_as2b, b_as2b):
    xdb = _tc_head(x, rbf, W_kj, b_kj, W_rbf1, W_rbf2, W_down)
    sbfe = _tc_sbf(sbf, W_sbf1, W_sbf2)

    kj3 = edge_idx_kj.astype(I32).reshape(NCHUNK, JB, 128)
    ji3 = edge_idx_ji.astype(I32).reshape(NCHUNK, JB, 128)
    sbfe3i = sbfe.reshape(2, NCHUNK, CH, 16)

    parts = _sc_scatter(kj3, ji3, xdb, sbfe3i)

    weights = [
        (W_bs1a, b_bs1a), (W_bs1b, b_bs1b), (W_bs2, b_bs2),
        (W_as1a, b_as1a), (W_as1b, b_as1b),
        (W_as2a, b_as2a), (W_as2b, b_as2b),
    ]
    return _tc_tail(x, parts, W_up, W_ji, b_ji, weights)
